# Initial kernel scaffold; baseline (speedup 1.0000x reference)
import jax
import jax.numpy as jnp
from jax.experimental import pallas as pl


def kernel(features, edge_index, W, b):
    raise NotImplementedError("write your pallas kernel here")


# speedup vs baseline: 5.2593x; 5.2593x over previous
import functools

import jax
import jax.numpy as jnp
from jax import lax
from jax.experimental import pallas as pl
from jax.experimental.pallas import tpu as pltpu
from jax.experimental.pallas import tpu_sc as plsc

N = 10000
NPAD = 10240
E = 320000
D = 128
CHUNK = 128
NCHUNK = E // CHUNK
NCORE = 2
NSUB = 16
NW = NCORE * NSUB
ROWS_PER_SUB = NPAD // NSUB
ROWBLK = 1000


def _sc_mesh():
    return plsc.VectorSubcoreMesh(core_axis_name="c", subcore_axis_name="s")


def _sc_degree(dst2, zerosD, onesD):
    @functools.partial(
        pl.kernel,
        out_type=jax.ShapeDtypeStruct((NCORE, NPAD, D), jnp.float32),
        mesh=_sc_mesh(),
        scratch_types=[
            pltpu.VMEM((1, CHUNK), jnp.int32),
            pltpu.VMEM((CHUNK, D), jnp.float32),
            pltpu.VMEM_SHARED((NPAD, D), jnp.float32),
        ],
    )
    def k(dst_hbm, z_hbm, ones_hbm, out_hbm, idx_v, ones_v, acc):
        cid = lax.axis_index("c")
        sid = lax.axis_index("s")
        gwid = sid * NCORE + cid
        pltpu.sync_copy(z_hbm, acc.at[pl.ds(sid * ROWS_PER_SUB, ROWS_PER_SUB)])
        pltpu.sync_copy(ones_hbm, ones_v)
        plsc.subcore_barrier()

        @pl.loop(gwid, NCHUNK, step=NW)
        def _(j):
            pltpu.sync_copy(dst_hbm.at[j], idx_v)
            pltpu.sync_copy(ones_v, acc.at[idx_v.at[0]], add=True)

        plsc.subcore_barrier()
        pltpu.sync_copy(
            acc.at[pl.ds(sid * ROWS_PER_SUB, ROWS_PER_SUB)],
            out_hbm.at[cid, pl.ds(sid * ROWS_PER_SUB, ROWS_PER_SUB)],
        )

    return k(dst2, zerosD, onesD)


def _sc_spmm(g, src2, dst2, zerosD):
    @functools.partial(
        pl.kernel,
        out_type=jax.ShapeDtypeStruct((NCORE, NPAD, D), jnp.float32),
        mesh=_sc_mesh(),
        scratch_types=[
            pltpu.VMEM((1, CHUNK), jnp.int32),
            pltpu.VMEM((1, CHUNK), jnp.int32),
            pltpu.VMEM((CHUNK, D), jnp.float32),
            pltpu.VMEM_SHARED((NPAD, D), jnp.float32),
        ],
    )
    def k(g_hbm, src_hbm, dst_hbm, z_hbm, out_hbm, sidx, didx, rows, acc):
        cid = lax.axis_index("c")
        sid = lax.axis_index("s")
        gwid = sid * NCORE + cid
        pltpu.sync_copy(z_hbm, acc.at[pl.ds(sid * ROWS_PER_SUB, ROWS_PER_SUB)])
        plsc.subcore_barrier()

        @pl.loop(gwid, NCHUNK, step=NW)
        def _(j):
            pltpu.sync_copy(src_hbm.at[j], sidx)
            pltpu.sync_copy(dst_hbm.at[j], didx)
            pltpu.sync_copy(g_hbm.at[sidx.at[0]], rows)
            pltpu.sync_copy(rows, acc.at[didx.at[0]], add=True)

        plsc.subcore_barrier()
        pltpu.sync_copy(
            acc.at[pl.ds(sid * ROWS_PER_SUB, ROWS_PER_SUB)],
            out_hbm.at[cid, pl.ds(sid * ROWS_PER_SUB, ROWS_PER_SUB)],
        )

    return k(g, src2, dst2, zerosD)


def _tc_norm_scale(degp, x):
    def body(dp_ref, x_ref, g0_ref, n_ref):
        deg = dp_ref[0] + dp_ref[1]
        nrm = lax.rsqrt(jnp.maximum(deg[:, 0:1], 1.0))
        n_ref[...] = jnp.broadcast_to(nrm, n_ref.shape)
        g0_ref[...] = x_ref[...] * nrm

    return pl.pallas_call(
        body,
        grid=(N // ROWBLK,),
        in_specs=[
            pl.BlockSpec((NCORE, ROWBLK, D), lambda i: (0, i, 0)),
            pl.BlockSpec((ROWBLK, D), lambda i: (i, 0)),
        ],
        out_specs=[
            pl.BlockSpec((ROWBLK, D), lambda i: (i, 0)),
            pl.BlockSpec((ROWBLK, 16), lambda i: (i, 0)),
        ],
        out_shape=[
            jax.ShapeDtypeStruct((N, D), jnp.float32),
            jax.ShapeDtypeStruct((N, 16), jnp.float32),
        ],
    )(degp, x)


def _tc_combine(parts, norm16):
    def body(p_ref, n_ref, o_ref):
        nr = n_ref[:, 0:1]
        o_ref[...] = (p_ref[0] + p_ref[1]) * (nr * nr)

    return pl.pallas_call(
        body,
        grid=(N // ROWBLK,),
        in_specs=[
            pl.BlockSpec((NCORE, ROWBLK, D), lambda i: (0, i, 0)),
            pl.BlockSpec((ROWBLK, 16), lambda i: (i, 0)),
        ],
        out_specs=pl.BlockSpec((ROWBLK, D), lambda i: (i, 0)),
        out_shape=jax.ShapeDtypeStruct((N, D), jnp.float32),
    )(parts, norm16)


def _tc_final(parts, norm16, W, b2):
    def body(p_ref, n_ref, w_ref, b_ref, o_ref):
        h = (p_ref[0] + p_ref[1]) * n_ref[:, 0:1]
        o_ref[...] = (
            jnp.dot(h, w_ref[...], preferred_element_type=jnp.float32)
            + b_ref[...]
        )

    return pl.pallas_call(
        body,
        grid=(N // ROWBLK,),
        in_specs=[
            pl.BlockSpec((NCORE, ROWBLK, D), lambda i: (0, i, 0)),
            pl.BlockSpec((ROWBLK, 16), lambda i: (i, 0)),
            pl.BlockSpec((D, D), lambda i: (0, 0)),
            pl.BlockSpec((1, D), lambda i: (0, 0)),
        ],
        out_specs=pl.BlockSpec((ROWBLK, D), lambda i: (i, 0)),
        out_shape=jax.ShapeDtypeStruct((N, D), jnp.float32),
    )(parts, norm16, W, b2)


@jax.jit
def kernel(features, edge_index, W, b):
    src2 = edge_index[0].reshape(NCHUNK, 1, CHUNK)
    dst2 = edge_index[1].reshape(NCHUNK, 1, CHUNK)
    onesD = jnp.ones((CHUNK, D), jnp.float32)
    zerosD = jnp.zeros((ROWS_PER_SUB, D), jnp.float32)
    b2 = b.reshape(1, D)

    degp = _sc_degree(dst2, zerosD, onesD)
    g, norm16 = _tc_norm_scale(degp, features)
    for hop in range(3):
        parts = _sc_spmm(g, src2, dst2, zerosD)
        if hop < 2:
            g = _tc_combine(parts, norm16)
    return _tc_final(parts, norm16, W, b2)
